# Initial kernel scaffold; baseline (speedup 1.0000x reference)
import jax
import jax.numpy as jnp
from jax.experimental import pallas as pl


def kernel(x, edge_index, edge_attr, W, W_e, attn_l, attn_r, attn_e, bias):
    raise NotImplementedError("write your pallas kernel here")


# speedup vs baseline: 26.2495x; 26.2495x over previous
import functools

import jax
import jax.numpy as jnp
from jax import lax
from jax.experimental import pallas as pl
from jax.experimental.pallas import tpu as pltpu
from jax.experimental.pallas import tpu_sc as plsc

N = 10000
E = 320000
D = 128
H = 4
DH = 32

NC, NS = 2, 16
NW = NC * NS
EPT = E // NW
C = 80
NCHUNK = EPT // C
N2 = 10240
ROWS = N2 // NS
AW = 144

BN = 2000
BE = 2000


def _node_body(x_ref, w_ref, al_ref, ar_ref, feat_ref, el_ref, er_ref):
    f = jnp.dot(x_ref[...], w_ref[...], preferred_element_type=jnp.float32)
    feat_ref[...] = f
    el_ref[...] = jnp.dot(f, al_ref[...], preferred_element_type=jnp.float32)
    er_ref[...] = jnp.dot(f, ar_ref[...], preferred_element_type=jnp.float32)


def _node_call(x, W, A_l, A_r):
    return pl.pallas_call(
        _node_body,
        grid=(N // BN,),
        in_specs=[
            pl.BlockSpec((BN, D), lambda i: (i, 0)),
            pl.BlockSpec((D, D), lambda i: (0, 0)),
            pl.BlockSpec((D, 16), lambda i: (0, 0)),
            pl.BlockSpec((D, 16), lambda i: (0, 0)),
        ],
        out_specs=[
            pl.BlockSpec((BN, D), lambda i: (i, 0)),
            pl.BlockSpec((BN, 16), lambda i: (i, 0)),
            pl.BlockSpec((BN, 16), lambda i: (i, 0)),
        ],
        out_shape=[
            jax.ShapeDtypeStruct((N, D), jnp.float32),
            jax.ShapeDtypeStruct((N, 16), jnp.float32),
            jax.ShapeDtypeStruct((N, 16), jnp.float32),
        ],
    )(x, W, A_l, A_r)


def _edge_body(ea_ref, we_ref, ae_ref, fe_ref, ee_ref):
    f = jnp.dot(ea_ref[...], we_ref[...], preferred_element_type=jnp.float32)
    fe_ref[...] = f
    ee_ref[...] = jnp.dot(f, ae_ref[...], preferred_element_type=jnp.float32)


def _edge_call(edge_attr, W_e, A_e):
    return pl.pallas_call(
        _edge_body,
        grid=(E // BE,),
        in_specs=[
            pl.BlockSpec((BE, D), lambda i: (i, 0)),
            pl.BlockSpec((D, D), lambda i: (0, 0)),
            pl.BlockSpec((D, 16), lambda i: (0, 0)),
        ],
        out_specs=[
            pl.BlockSpec((BE, D), lambda i: (i, 0)),
            pl.BlockSpec((BE, 16), lambda i: (i, 0)),
        ],
        out_shape=[
            jax.ShapeDtypeStruct((E, D), jnp.float32),
            jax.ShapeDtypeStruct((E, 16), jnp.float32),
        ],
    )(edge_attr, W_e, A_e)


def _sc_agg_body(src_hbm, dst_hbm, elt_hbm, ert_hbm, eep_hbm, feat_hbm, fe_hbm,
                 out_hbm, src_v, dst_v, elg, erg, eev, fg, fev, msg,
                 acc_sh, s1, s2, s3):
    cid = lax.axis_index("c")
    sid = lax.axis_index("s")
    wid = sid * NC + cid

    def zero_row(c, carry):
        for q in range(AW // 16):
            msg[c, pl.ds(q * 16, 16)] = jnp.zeros((16,), jnp.float32)
        return carry

    lax.fori_loop(0, C, zero_row, 0, unroll=False)
    for j in range(ROWS // C):
        pltpu.sync_copy(msg, acc_sh.at[pl.ds(sid * ROWS + j * C, C)])
    plsc.subcore_barrier()

    def chunk(k, carry):
        base = pl.multiple_of(wid * EPT + k * C, 8)
        pltpu.sync_copy(src_hbm.at[pl.ds(base, C)], src_v)
        pltpu.sync_copy(dst_hbm.at[pl.ds(base, C)], dst_v)
        pltpu.sync_copy(eep_hbm.at[pl.ds(base, C)], eev)
        cp1 = pltpu.async_copy(elt_hbm.at[src_v], elg, s1)
        cp2 = pltpu.async_copy(ert_hbm.at[dst_v], erg, s2)
        cp3 = pltpu.async_copy(feat_hbm.at[src_v], fg, s3)
        pltpu.sync_copy(fe_hbm.at[pl.ds(base, C)], fev)
        cp1.wait()
        cp2.wait()
        cp3.wait()

        def edge(c, carry2):
            v = elg[c, :] + erg[c, :] + eev[c, :]
            lg = jnp.where(v >= 0.0, v, 0.2 * v)
            p = jnp.exp(lg)
            msg[c, pl.ds(D, 16)] = p
            for h in range(H):
                ps = p[h]
                for q in range(2):
                    sl = pl.ds(h * DH + q * 16, 16)
                    msg[c, sl] = ps * (fg[c, sl] + fev[c, sl])
            return carry2

        lax.fori_loop(0, C, edge, 0, unroll=False)
        pltpu.sync_copy(msg, acc_sh.at[dst_v], add=True)
        return carry

    lax.fori_loop(0, NCHUNK, chunk, 0, unroll=False)
    plsc.subcore_barrier()
    for j in range(ROWS // C):
        pltpu.sync_copy(acc_sh.at[pl.ds(sid * ROWS + j * C, C)], msg)
        pltpu.sync_copy(msg, out_hbm.at[cid, pl.ds(sid * ROWS + j * C, C)])


@functools.cache
def _sc_agg():
    return functools.partial(
        pl.kernel,
        out_type=jax.ShapeDtypeStruct((NC, N2, AW), jnp.float32),
        mesh=plsc.VectorSubcoreMesh(core_axis_name="c", subcore_axis_name="s",
                                    num_cores=NC, num_subcores=NS),
        compiler_params=pltpu.CompilerParams(use_tc_tiling_on_sc=False),
        scratch_types=[
            pltpu.VMEM((C,), jnp.int32),
            pltpu.VMEM((C,), jnp.int32),
            pltpu.VMEM((C, 16), jnp.float32),
            pltpu.VMEM((C, 16), jnp.float32),
            pltpu.VMEM((C, 16), jnp.float32),
            pltpu.VMEM((C, D), jnp.float32),
            pltpu.VMEM((C, D), jnp.float32),
            pltpu.VMEM((C, AW), jnp.float32),
            pltpu.VMEM_SHARED((N2, AW), jnp.float32),
            pltpu.SemaphoreType.DMA,
            pltpu.SemaphoreType.DMA,
            pltpu.SemaphoreType.DMA,
        ],
    )(_sc_agg_body)


def _fin_body(a0_ref, a1_ref, x_ref, b_ref, s_ref, o_ref):
    a = a0_ref[...] + a1_ref[...]
    msgs = a[:, :D]
    den16 = a[:, D:]
    den = jnp.dot(den16, s_ref[...], preferred_element_type=jnp.float32)
    r = jnp.where(den > 0.0, msgs / den, 0.0)
    o_ref[...] = jnp.maximum(r + x_ref[...] + b_ref[...], 0.0)


def _fin_call(a0, a1, x, bias2d, S):
    return pl.pallas_call(
        _fin_body,
        grid=(N // BN,),
        in_specs=[
            pl.BlockSpec((BN, AW), lambda i: (i, 0)),
            pl.BlockSpec((BN, AW), lambda i: (i, 0)),
            pl.BlockSpec((BN, D), lambda i: (i, 0)),
            pl.BlockSpec((1, D), lambda i: (0, 0)),
            pl.BlockSpec((16, D), lambda i: (0, 0)),
        ],
        out_specs=pl.BlockSpec((BN, D), lambda i: (i, 0)),
        out_shape=jax.ShapeDtypeStruct((N, D), jnp.float32),
    )(a0, a1, x, bias2d, S)


def _placement(attn):
    rows = jnp.arange(D)
    cols = jnp.repeat(jnp.arange(H), DH)
    return jnp.zeros((D, 16), jnp.float32).at[rows, cols].set(attn.reshape(-1))


def kernel(x, edge_index, edge_attr, W, W_e, attn_l, attn_r, attn_e, bias):
    src = edge_index[0]
    dst = edge_index[1]

    A_l = _placement(attn_l)
    A_r = _placement(attn_r)
    A_e = _placement(attn_e)
    rows = jnp.arange(D)
    cols = jnp.repeat(jnp.arange(H), DH)
    S = jnp.zeros((16, D), jnp.float32).at[cols, rows].set(1.0)

    feat, el_t, er_t = _node_call(x, W, A_l, A_r)
    feat_e, ee_p = _edge_call(edge_attr, W_e, A_e)

    acc = _sc_agg()(src, dst, el_t, er_t, ee_p, feat, feat_e)

    return _fin_call(acc[0], acc[1], x, bias.reshape(1, D), S)


# speedup vs baseline: 30.8055x; 1.1736x over previous
import functools

import jax
import jax.numpy as jnp
from jax import lax
from jax.experimental import pallas as pl
from jax.experimental.pallas import tpu as pltpu
from jax.experimental.pallas import tpu_sc as plsc

N = 10000
E = 320000
D = 128
H = 4
DH = 32

NC, NS = 2, 16
NW = NC * NS
EPT = E // NW
C = 40
NCHUNK = EPT // C
N2 = 10240
ROWS = N2 // NS
AW = 144

BN = 2000
BE = 2000


def _node_body(x_ref, w_ref, al_ref, ar_ref, feat_ref, el_ref, er_ref):
    f = jnp.dot(x_ref[...], w_ref[...], preferred_element_type=jnp.float32)
    feat_ref[...] = f
    el_ref[...] = jnp.dot(f, al_ref[...], preferred_element_type=jnp.float32)
    er_ref[...] = jnp.dot(f, ar_ref[...], preferred_element_type=jnp.float32)


def _node_call(x, W, A_l, A_r):
    return pl.pallas_call(
        _node_body,
        grid=(N // BN,),
        in_specs=[
            pl.BlockSpec((BN, D), lambda i: (i, 0)),
            pl.BlockSpec((D, D), lambda i: (0, 0)),
            pl.BlockSpec((D, 16), lambda i: (0, 0)),
            pl.BlockSpec((D, 16), lambda i: (0, 0)),
        ],
        out_specs=[
            pl.BlockSpec((BN, D), lambda i: (i, 0)),
            pl.BlockSpec((BN, 16), lambda i: (i, 0)),
            pl.BlockSpec((BN, 16), lambda i: (i, 0)),
        ],
        out_shape=[
            jax.ShapeDtypeStruct((N, D), jnp.float32),
            jax.ShapeDtypeStruct((N, 16), jnp.float32),
            jax.ShapeDtypeStruct((N, 16), jnp.float32),
        ],
    )(x, W, A_l, A_r)


def _edge_body(ea_ref, we_ref, ae_ref, fe_ref, ee_ref):
    f = jnp.dot(ea_ref[...], we_ref[...], preferred_element_type=jnp.float32)
    fe_ref[...] = f
    ee_ref[...] = jnp.dot(f, ae_ref[...], preferred_element_type=jnp.float32)


def _edge_call(edge_attr, W_e, A_e):
    return pl.pallas_call(
        _edge_body,
        grid=(E // BE,),
        in_specs=[
            pl.BlockSpec((BE, D), lambda i: (i, 0)),
            pl.BlockSpec((D, D), lambda i: (0, 0)),
            pl.BlockSpec((D, 16), lambda i: (0, 0)),
        ],
        out_specs=[
            pl.BlockSpec((BE, D), lambda i: (i, 0)),
            pl.BlockSpec((BE, 16), lambda i: (i, 0)),
        ],
        out_shape=[
            jax.ShapeDtypeStruct((E, D), jnp.float32),
            jax.ShapeDtypeStruct((E, 16), jnp.float32),
        ],
    )(edge_attr, W_e, A_e)


def _sc_agg_body(src_hbm, dst_hbm, elt_hbm, ert_hbm, eep_hbm, feat_hbm, fe_hbm,
                 out_hbm,
                 src_v0, dst_v0, elg0, erg0, eev0, fg0, fev0,
                 src_v1, dst_v1, elg1, erg1, eev1, fg1, fev1,
                 msg0, acc_sh, sl0, sl1, sg0, sg1):
    cid = lax.axis_index("c")
    sid = lax.axis_index("s")
    wid = sid * NC + cid
    bufs = ((src_v0, dst_v0, elg0, erg0, eev0, fg0, fev0, msg0, sl0, sg0),
            (src_v1, dst_v1, elg1, erg1, eev1, fg1, fev1, msg0, sl1, sg1))

    def zero_row(c, carry):
        for q in range(AW // 16):
            msg0[c, pl.ds(q * 16, 16)] = jnp.zeros((16,), jnp.float32)
        return carry

    lax.fori_loop(0, C, zero_row, 0, unroll=False)
    for j in range(ROWS // C):
        pltpu.sync_copy(msg0, acc_sh.at[pl.ds(sid * ROWS + j * C, C)])
    plsc.subcore_barrier()

    def _base(k):
        return pl.multiple_of(wid * EPT + k * C, 8)

    def lin_descs(k, b):
        src_v, dst_v, elg, erg, eev, fg, fev, msg, sl, sg = bufs[b]
        base = _base(k)
        return ((src_hbm.at[pl.ds(base, C)], src_v, sl),
                (dst_hbm.at[pl.ds(base, C)], dst_v, sl),
                (eep_hbm.at[pl.ds(base, C)], eev, sl),
                (fe_hbm.at[pl.ds(base, C)], fev, sl))

    def gth_descs(b):
        src_v, dst_v, elg, erg, eev, fg, fev, msg, sl, sg = bufs[b]
        return ((elt_hbm.at[src_v], elg, sg),
                (ert_hbm.at[dst_v], erg, sg),
                (feat_hbm.at[src_v], fg, sg))

    def issue_lin(k, b):
        for d in lin_descs(k, b):
            pltpu.async_copy(*d)

    def wait_lin(k, b):
        for d in lin_descs(k, b):
            pltpu.make_async_copy(*d).wait()

    def issue_gth(b):
        for d in gth_descs(b):
            pltpu.async_copy(*d)

    def wait_gth(b):
        for d in gth_descs(b):
            pltpu.make_async_copy(*d).wait()

    def compute(b):
        src_v, dst_v, elg, erg, eev, fg, fev, msg, sl, sg = bufs[b]

        def edge(c, carry2):
            v = elg[c, :] + erg[c, :] + eev[c, :]
            lg = jnp.where(v >= 0.0, v, 0.2 * v)
            p = jnp.exp(lg)
            msg[c, pl.ds(D, 16)] = p
            for h in range(H):
                ps = p[h]
                for q in range(2):
                    sl2 = pl.ds(h * DH + q * 16, 16)
                    msg[c, sl2] = ps * (fg[c, sl2] + fev[c, sl2])
            return carry2

        lax.fori_loop(0, C, edge, 0, unroll=False)
        pltpu.sync_copy(msg, acc_sh.at[dst_v], add=True)

    issue_lin(0, 0)
    wait_lin(0, 0)
    issue_gth(0)
    issue_lin(1, 1)

    def pair(g, carry):
        e = 2 * g
        o = e + 1
        wait_lin(o, 1)
        issue_gth(1)
        wait_gth(0)
        compute(0)
        issue_lin(e + 2, 0)
        wait_gth(1)
        compute(1)
        wait_lin(e + 2, 0)
        issue_gth(0)
        issue_lin(o + 2, 1)
        return carry

    lax.fori_loop(0, NCHUNK // 2 - 1, pair, 0, unroll=False)
    wait_lin(NCHUNK - 1, 1)
    issue_gth(1)
    wait_gth(0)
    compute(0)
    wait_gth(1)
    compute(1)

    plsc.subcore_barrier()
    for j in range(ROWS // C):
        pltpu.sync_copy(acc_sh.at[pl.ds(sid * ROWS + j * C, C)], msg0)
        pltpu.sync_copy(msg0, out_hbm.at[cid, pl.ds(sid * ROWS + j * C, C)])


@functools.cache
def _sc_agg():
    return functools.partial(
        pl.kernel,
        out_type=jax.ShapeDtypeStruct((NC, N2, AW), jnp.float32),
        mesh=plsc.VectorSubcoreMesh(core_axis_name="c", subcore_axis_name="s",
                                    num_cores=NC, num_subcores=NS),
        compiler_params=pltpu.CompilerParams(use_tc_tiling_on_sc=False),
        scratch_types=[
            pltpu.VMEM((C,), jnp.int32),
            pltpu.VMEM((C,), jnp.int32),
            pltpu.VMEM((C, 16), jnp.float32),
            pltpu.VMEM((C, 16), jnp.float32),
            pltpu.VMEM((C, 16), jnp.float32),
            pltpu.VMEM((C, D), jnp.float32),
            pltpu.VMEM((C, D), jnp.float32),
            pltpu.VMEM((C,), jnp.int32),
            pltpu.VMEM((C,), jnp.int32),
            pltpu.VMEM((C, 16), jnp.float32),
            pltpu.VMEM((C, 16), jnp.float32),
            pltpu.VMEM((C, 16), jnp.float32),
            pltpu.VMEM((C, D), jnp.float32),
            pltpu.VMEM((C, D), jnp.float32),
            pltpu.VMEM((C, AW), jnp.float32),
            pltpu.VMEM_SHARED((N2, AW), jnp.float32),
            pltpu.SemaphoreType.DMA,
            pltpu.SemaphoreType.DMA,
            pltpu.SemaphoreType.DMA,
            pltpu.SemaphoreType.DMA,
        ],
    )(_sc_agg_body)


def _fin_body(a0_ref, a1_ref, x_ref, b_ref, s_ref, o_ref):
    a = a0_ref[...] + a1_ref[...]
    msgs = a[:, :D]
    den16 = a[:, D:]
    den = jnp.dot(den16, s_ref[...], preferred_element_type=jnp.float32)
    r = jnp.where(den > 0.0, msgs / den, 0.0)
    o_ref[...] = jnp.maximum(r + x_ref[...] + b_ref[...], 0.0)


def _fin_call(a0, a1, x, bias2d, S):
    return pl.pallas_call(
        _fin_body,
        grid=(N // BN,),
        in_specs=[
            pl.BlockSpec((BN, AW), lambda i: (i, 0)),
            pl.BlockSpec((BN, AW), lambda i: (i, 0)),
            pl.BlockSpec((BN, D), lambda i: (i, 0)),
            pl.BlockSpec((1, D), lambda i: (0, 0)),
            pl.BlockSpec((16, D), lambda i: (0, 0)),
        ],
        out_specs=pl.BlockSpec((BN, D), lambda i: (i, 0)),
        out_shape=jax.ShapeDtypeStruct((N, D), jnp.float32),
    )(a0, a1, x, bias2d, S)


def _placement(attn):
    rows = jnp.arange(D)
    cols = jnp.repeat(jnp.arange(H), DH)
    return jnp.zeros((D, 16), jnp.float32).at[rows, cols].set(attn.reshape(-1))


def kernel(x, edge_index, edge_attr, W, W_e, attn_l, attn_r, attn_e, bias):
    src = edge_index[0]
    dst = edge_index[1]

    A_l = _placement(attn_l)
    A_r = _placement(attn_r)
    A_e = _placement(attn_e)
    rows = jnp.arange(D)
    cols = jnp.repeat(jnp.arange(H), DH)
    S = jnp.zeros((16, D), jnp.float32).at[cols, rows].set(1.0)

    feat, el_t, er_t = _node_call(x, W, A_l, A_r)
    feat_e, ee_p = _edge_call(edge_attr, W_e, A_e)

    acc = _sc_agg()(src, dst, el_t, er_t, ee_p, feat, feat_e)

    return _fin_call(acc[0], acc[1], x, bias.reshape(1, D), S)
